# baseline (speedup 1.0000x reference)
import functools

import jax
import jax.numpy as jnp
from jax import lax
from jax.experimental import pallas as pl
from jax.experimental.pallas import tpu as pltpu
from jax.experimental.pallas import tpu_sc as plsc

NUM_STEPS = 1000
DIM = 128
BATCH = 16384


def _mlp_body(emb_ref, w1_ref, b1_ref, w2_ref, b2_ref, out_ref):
    dn = (((1,), (1,)), ((), ()))
    x = lax.dot_general(emb_ref[...], w1_ref[...], dn,
                        preferred_element_type=jnp.float32)
    x = x + b1_ref[...]
    x = x * jax.nn.sigmoid(x)
    x = lax.dot_general(x, w2_ref[...], dn, preferred_element_type=jnp.float32)
    x = x + b2_ref[...]
    out_ref[...] = x * jax.nn.sigmoid(x)


def _mlp_table(embedding, w1, b1, w2, b2):
    n = embedding.shape[0]
    return pl.pallas_call(
        _mlp_body,
        out_shape=jax.ShapeDtypeStruct((n, DIM), jnp.float32),
    )(embedding, w1, b1, w2, b2)


@functools.lru_cache(maxsize=None)
def _make_gather():
    info = plsc.get_sparse_core_info()
    nc, ns = info.num_cores, info.num_subcores
    nw = nc * ns
    rows_per_w = BATCH // nw

    half = rows_per_w // 2

    def _gather_body(table_hbm, idx_hbm, out_hbm, idx_v, rows_v, g0, g1, osem):
        wid = lax.axis_index("s") * nc + lax.axis_index("c")
        base = wid * rows_per_w
        pltpu.sync_copy(idx_hbm.at[pl.ds(base, rows_per_w)], idx_v)
        c0 = pltpu.async_copy(table_hbm.at[idx_v.at[pl.ds(0, half)]],
                              rows_v.at[pl.ds(0, half)], g0)
        c1 = pltpu.async_copy(table_hbm.at[idx_v.at[pl.ds(half, half)]],
                              rows_v.at[pl.ds(half, half)], g1)
        c0.wait()
        o0 = pltpu.async_copy(rows_v.at[pl.ds(0, half)],
                              out_hbm.at[pl.ds(base, half)], osem)
        c1.wait()
        o1 = pltpu.async_copy(rows_v.at[pl.ds(half, half)],
                              out_hbm.at[pl.ds(base + half, half)], osem)
        o0.wait()
        o1.wait()

    return pl.kernel(
        _gather_body,
        out_type=jax.ShapeDtypeStruct((BATCH, DIM), jnp.float32),
        mesh=plsc.VectorSubcoreMesh(core_axis_name="c", subcore_axis_name="s"),
        scratch_types=[
            pltpu.VMEM((rows_per_w,), jnp.int32),
            pltpu.VMEM((rows_per_w, DIM), jnp.float32),
            pltpu.SemaphoreType.DMA,
            pltpu.SemaphoreType.DMA,
            pltpu.SemaphoreType.DMA,
        ],
    )


def kernel(diffusion_step, embedding, W1, b1, W2, b2):
    table = _mlp_table(embedding, W1, b1.reshape(1, DIM), W2, b2.reshape(1, DIM))
    idx = diffusion_step.astype(jnp.int32)
    return _make_gather()(table, idx)


# speedup vs baseline: 1.0167x; 1.0167x over previous
import functools

import jax
import jax.numpy as jnp
from jax import lax
from jax.experimental import pallas as pl
from jax.experimental.pallas import tpu as pltpu
from jax.experimental.pallas import tpu_sc as plsc

NUM_STEPS = 1000
DIM = 128
BATCH = 16384


def _mlp_body(emb_ref, w1_ref, b1_ref, w2_ref, b2_ref, out_ref):
    dn = (((1,), (1,)), ((), ()))
    x = lax.dot_general(emb_ref[...], w1_ref[...], dn,
                        preferred_element_type=jnp.float32)
    x = x + b1_ref[...]
    x = x * jax.nn.sigmoid(x)
    x = lax.dot_general(x, w2_ref[...], dn, preferred_element_type=jnp.float32)
    x = x + b2_ref[...]
    out_ref[...] = x * jax.nn.sigmoid(x)


def _mlp_table(embedding, w1, b1, w2, b2):
    n = embedding.shape[0]
    return pl.pallas_call(
        _mlp_body,
        out_shape=jax.ShapeDtypeStruct((n, DIM), jnp.float32),
    )(embedding, w1, b1, w2, b2)


@functools.lru_cache(maxsize=None)
def _make_gather():
    info = plsc.get_sparse_core_info()
    nc, ns = info.num_cores, info.num_subcores
    nw = nc * ns
    rows_per_w = BATCH // nw

    def _gather_body(table_hbm, idx_hbm, out_hbm, idx_v, rows_v, gsem):
        wid = lax.axis_index("s") * nc + lax.axis_index("c")
        base = wid * rows_per_w
        pltpu.sync_copy(idx_hbm.at[pl.ds(base, rows_per_w)], idx_v)
        pltpu.async_copy(table_hbm.at[idx_v], rows_v, gsem).wait()
        pltpu.sync_copy(rows_v, out_hbm.at[pl.ds(base, rows_per_w)])

    return pl.kernel(
        _gather_body,
        out_type=jax.ShapeDtypeStruct((BATCH, DIM), jnp.float32),
        mesh=plsc.VectorSubcoreMesh(core_axis_name="c", subcore_axis_name="s"),
        scratch_types=[
            pltpu.VMEM((rows_per_w,), jnp.int32),
            pltpu.VMEM((rows_per_w, DIM), jnp.float32),
            pltpu.SemaphoreType.DMA,
        ],
    )


def kernel(diffusion_step, embedding, W1, b1, W2, b2):
    table = _mlp_table(embedding, W1, b1.reshape(1, DIM), W2, b2.reshape(1, DIM))
    idx = diffusion_step.astype(jnp.int32)
    return _make_gather()(table, idx)
